# baseline (speedup 1.0000x reference)
import jax
import jax.numpy as jnp
from jax.experimental import pallas as pl
from jax.experimental.pallas import tpu as pltpu

_GAMMA = 2.0
_SC = 0.0001
_ALPHA = 0.25

_N = 20000
_C = 80
_G = 128
_BN = 1000
_NB = _N // _BN


def _mask_loss_kernel(oh_ref, logits_ref, scores_ref, iib_ref, iou_ref,
                      out_ref, al, pw, ab, conf, colmax, nega, posa, bnega):
    s = pl.program_id(0)
    i = pl.program_id(1)

    @pl.when((s == 0) & (i == 0))
    def _init():
        conf[...] = jnp.zeros_like(conf)
        colmax[...] = jnp.zeros_like(colmax)
        nega[...] = jnp.zeros_like(nega)
        posa[...] = jnp.zeros_like(posa)
        bnega[...] = jnp.zeros_like(bnega)

    @pl.when(s == 0)
    def _stage0():
        lb = logits_ref[...]
        bp = jnp.clip(jax.nn.sigmoid(lb), _SC, 1.0 - _SC)
        l1m = jnp.log(1.0 - bp)
        fneg = -l1m * bp * bp
        iib = iib_ref[...]
        iibf = iib.astype(jnp.float32)
        mask = iib > 0
        kk = jnp.sum(iibf, axis=1, keepdims=True)
        any_in = kk > 0.0
        p0 = bp[:, 0:1]
        l1m0 = l1m[:, 0:1]
        b = l1m0 * p0 * p0
        logp0 = jnp.clip(lb[:, 0:1] + l1m0,
                         jnp.log(_SC), jnp.log(1.0 - _SC))
        a = logp0 * (1.0 - p0) * (1.0 - p0)
        nega[...] += (jnp.sum(fneg, keepdims=True).reshape(1, 1)
                      + jnp.sum(jnp.where(any_in, b, 0.0),
                                keepdims=True).reshape(1, 1))
        ab[pl.ds(i * _BN, _BN), 0:1] = a
        ab[pl.ds(i * _BN, _BN), 1:2] = b
        align = jnp.dot(scores_ref[...], oh_ref[...],
                        preferred_element_type=jnp.float32) * iibf
        am = jnp.where(mask, align, 0.0)
        al[pl.ds(i * _BN, _BN), :] = jnp.where(mask, align, -1.0)
        conf[...] = jnp.maximum(conf[...], jnp.max(am, axis=0, keepdims=True))

    @pl.when(s == 1)
    def _stage1():
        giou = jnp.dot(iou_ref[...], oh_ref[...],
                       preferred_element_type=jnp.float32)
        alb = al[pl.ds(i * _BN, _BN), :]
        maskb = alb >= 0.0
        base = jnp.where(alb > 0.0, alb, 1.0)
        cf = conf[...]
        p = jnp.exp(cf * jnp.log(base))
        p = jnp.where((alb == 0.0) & (cf > 0.0), 0.0, p)
        pwb = jnp.where(maskb, p * giou, 0.0)
        pw[pl.ds(i * _BN, _BN), :] = pwb
        colmax[...] = jnp.maximum(colmax[...],
                                  jnp.max(pwb, axis=0, keepdims=True))

    @pl.when(s == 2)
    def _stage2():
        pwb = pw[pl.ds(i * _BN, _BN), :]
        alb = al[pl.ds(i * _BN, _BN), :]
        maskb = alb >= 0.0
        rinv = 1.0 / (colmax[...] + _SC)
        t = jnp.clip((pwb + _SC) * rinv, _SC, 1.0 - _SC)
        t = jnp.where(maskb, t, 0.0)
        s1 = jnp.sum(t, axis=1, keepdims=True)
        s2 = jnp.sum(t * t, axis=1, keepdims=True)
        kk = jnp.sum(maskb.astype(jnp.float32), axis=1, keepdims=True)
        a = ab[pl.ds(i * _BN, _BN), 0:1]
        b = ab[pl.ds(i * _BN, _BN), 1:2]
        posa[...] += jnp.sum(a * s2, keepdims=True).reshape(1, 1)
        bnega[...] += jnp.sum(b * (kk - 2.0 * s1 + s2),
                              keepdims=True).reshape(1, 1)

    @pl.when((s == 2) & (i == _NB - 1))
    def _final():
        total = ((1.0 - _ALPHA) * nega[...]
                 - _ALPHA * posa[...] - _ALPHA * bnega[...])
        out_ref[...] = jnp.broadcast_to(total, out_ref.shape)


def kernel(logits_pred, scores, iou_map, is_in_boxes, labels, num_pos_avg):
    oh = (labels[None, :] ==
          jax.lax.broadcasted_iota(jnp.int32, (_C, _G), 0)).astype(jnp.float32)
    out = pl.pallas_call(
        _mask_loss_kernel,
        grid=(3, _NB),
        in_specs=[
            pl.BlockSpec((_C, _G), lambda s, i: (0, 0)),
            pl.BlockSpec((_BN, _C), lambda s, i: (jnp.where(s == 0, i, 0), 0)),
            pl.BlockSpec((_BN, _C), lambda s, i: (jnp.where(s == 0, i, 0), 0)),
            pl.BlockSpec((_BN, _G), lambda s, i: (jnp.where(s == 0, i, 0), 0)),
            pl.BlockSpec((_BN, _C), lambda s, i: (jnp.where(s == 1, i, 0), 0)),
        ],
        out_specs=pl.BlockSpec((8, 128), lambda s, i: (0, 0)),
        out_shape=jax.ShapeDtypeStruct((8, 128), jnp.float32),
        scratch_shapes=[
            pltpu.VMEM((_N, _G), jnp.float32),
            pltpu.VMEM((_N, _G), jnp.float32),
            pltpu.VMEM((_N, 2), jnp.float32),
            pltpu.VMEM((1, _G), jnp.float32),
            pltpu.VMEM((1, _G), jnp.float32),
            pltpu.VMEM((1, 1), jnp.float32),
            pltpu.VMEM((1, 1), jnp.float32),
            pltpu.VMEM((1, 1), jnp.float32),
        ],
        compiler_params=pltpu.CompilerParams(
            dimension_semantics=("arbitrary", "arbitrary")),
    )(oh, logits_pred, scores, is_in_boxes, iou_map)
    return out[0, 0] / num_pos_avg


# speedup vs baseline: 1.3175x; 1.3175x over previous
import jax
import jax.numpy as jnp
from jax import lax
from jax.experimental import pallas as pl
from jax.experimental.pallas import tpu as pltpu

_GAMMA = 2.0
_SC = 0.0001
_ALPHA = 0.25

_N = 20000
_C = 80
_G = 128
_BN = 1000
_NB = _N // _BN

_DN_STD = (((1,), (0,)), ((), ()))
_DN_TT = (((0,), (0,)), ((), ()))


def _mask_loss_kernel(oh_ref, logits_ref, scores_ref, iib_ref, iou_ref,
                      out_ref, la, pw, ab, conf, colmax, negacc, coracc,
                      acca, accb):
    s = pl.program_id(0)
    i = pl.program_id(1)

    @pl.when((s == 0) & (i == 0))
    def _init():
        conf[...] = jnp.zeros_like(conf)
        colmax[...] = jnp.zeros_like(colmax)
        negacc[...] = jnp.zeros_like(negacc)
        coracc[...] = jnp.zeros_like(coracc)
        acca[...] = jnp.zeros_like(acca)
        accb[...] = jnp.zeros_like(accb)

    @pl.when(s == 0)
    def _stage0():
        onesr = jnp.ones((1, _BN), jnp.float32)
        lb = logits_ref[...]
        bp = jnp.clip(jax.nn.sigmoid(lb), _SC, 1.0 - _SC)
        onem = 1.0 - bp
        l1m = jnp.log(onem)
        fneg = -l1m * bp * bp
        negacc[...] += lax.dot_general(
            onesr, fneg, _DN_STD, preferred_element_type=jnp.float32)
        iib = iib_ref[...]
        iibf = iib.astype(jnp.float32)
        mask = iib > 0
        kk = lax.dot_general(iibf, jnp.ones((_G, 1), jnp.float32), _DN_STD,
                             preferred_element_type=jnp.float32)
        b = -fneg[:, 0:1]
        logp0 = jnp.clip(lb[:, 0:1] + l1m[:, 0:1],
                         jnp.log(_SC), jnp.log(1.0 - _SC))
        a = logp0 * onem[:, 0:1] * onem[:, 0:1]
        selb = jnp.where(kk > 0.0, b, 0.0)
        coracc[...] += lax.dot_general(
            onesr, selb, _DN_STD, preferred_element_type=jnp.float32)
        ab[pl.ds(i * _BN, _BN), 0:1] = a
        ab[pl.ds(i * _BN, _BN), 1:2] = b
        align = lax.dot_general(scores_ref[...], oh_ref[...], _DN_STD,
                                preferred_element_type=jnp.float32) * iibf
        conf[...] = jnp.maximum(conf[...],
                                jnp.max(align, axis=0, keepdims=True))
        la[pl.ds(i * _BN, _BN), :] = jnp.where(
            mask, jnp.log(jnp.maximum(align, 1e-38)), 1.0)

    @pl.when(s == 1)
    def _stage1():
        giou = lax.dot_general(iou_ref[...], oh_ref[...], _DN_STD,
                               preferred_element_type=jnp.float32)
        lab = la[pl.ds(i * _BN, _BN), :]
        maskb = lab < 0.5
        p = jnp.exp(conf[...] * lab)
        pwb = jnp.where(maskb, p * giou, -1.0)
        pw[pl.ds(i * _BN, _BN), :] = pwb
        colmax[...] = jnp.maximum(colmax[...],
                                  jnp.max(pwb, axis=0, keepdims=True))

    @pl.when(s == 2)
    def _stage2():
        pwb = pw[pl.ds(i * _BN, _BN), :]
        maskb = pwb >= 0.0
        rinv = 1.0 / (colmax[...] + _SC)
        t = jnp.clip((pwb + _SC) * rinv, _SC, 1.0 - _SC)
        t = jnp.where(maskb, t, 0.0)
        u2 = t * t
        onemt = 1.0 - t
        w = jnp.where(maskb, onemt * onemt, 0.0)
        abb = ab[pl.ds(i * _BN, _BN), :]
        acca[...] += lax.dot_general(
            abb, u2, _DN_TT, preferred_element_type=jnp.float32)
        accb[...] += lax.dot_general(
            abb, w, _DN_TT, preferred_element_type=jnp.float32)

    @pl.when((s == 2) & (i == _NB - 1))
    def _final():
        sneg = jnp.sum(negacc[...], axis=1, keepdims=True)
        spos = jnp.sum(acca[0:1, :], axis=1, keepdims=True)
        sbneg = jnp.sum(accb[1:2, :], axis=1, keepdims=True)
        total = ((1.0 - _ALPHA) * (sneg + coracc[...])
                 - _ALPHA * (spos + sbneg))
        out_ref[...] = jnp.broadcast_to(total, out_ref.shape)


def kernel(logits_pred, scores, iou_map, is_in_boxes, labels, num_pos_avg):
    oh = (labels[None, :] ==
          jax.lax.broadcasted_iota(jnp.int32, (_C, _G), 0)).astype(jnp.float32)
    out = pl.pallas_call(
        _mask_loss_kernel,
        grid=(3, _NB),
        in_specs=[
            pl.BlockSpec((_C, _G), lambda s, i: (0, 0)),
            pl.BlockSpec((_BN, _C), lambda s, i: (jnp.where(s == 0, i, 0), 0)),
            pl.BlockSpec((_BN, _C), lambda s, i: (jnp.where(s == 0, i, 0), 0)),
            pl.BlockSpec((_BN, _G), lambda s, i: (jnp.where(s == 0, i, 0), 0)),
            pl.BlockSpec((_BN, _C), lambda s, i: (jnp.where(s == 1, i, 0), 0)),
        ],
        out_specs=pl.BlockSpec((8, 128), lambda s, i: (0, 0)),
        out_shape=jax.ShapeDtypeStruct((8, 128), jnp.float32),
        scratch_shapes=[
            pltpu.VMEM((_N, _G), jnp.float32),
            pltpu.VMEM((_N, _G), jnp.float32),
            pltpu.VMEM((_N, 2), jnp.float32),
            pltpu.VMEM((1, _G), jnp.float32),
            pltpu.VMEM((1, _G), jnp.float32),
            pltpu.VMEM((1, _C), jnp.float32),
            pltpu.VMEM((1, 1), jnp.float32),
            pltpu.VMEM((2, _G), jnp.float32),
            pltpu.VMEM((2, _G), jnp.float32),
        ],
        compiler_params=pltpu.CompilerParams(
            dimension_semantics=("arbitrary", "arbitrary")),
    )(oh, logits_pred, scores, is_in_boxes, iou_map)
    return out[0, 0] / num_pos_avg


# speedup vs baseline: 1.4247x; 1.0814x over previous
import jax
import jax.numpy as jnp
from jax import lax
from jax.experimental import pallas as pl
from jax.experimental.pallas import tpu as pltpu

_GAMMA = 2.0
_SC = 0.0001
_ALPHA = 0.25

_N = 20000
_C = 80
_G = 128
_BN = 1000
_NB = _N // _BN

_DN_STD = (((1,), (0,)), ((), ()))
_DN_TT = (((0,), (0,)), ((), ()))


def _mask_loss_kernel(oh_ref, logits_ref, scores_ref, iib_ref, iou_ref,
                      out_ref, la, ab, conf, macc, negacc, coracc,
                      acc1, acc2, acc3):
    s = pl.program_id(0)
    i = pl.program_id(1)

    @pl.when((s == 0) & (i == 0))
    def _init():
        conf[...] = jnp.zeros_like(conf)
        macc[...] = jnp.zeros_like(macc)
        negacc[...] = jnp.zeros_like(negacc)
        coracc[...] = jnp.zeros_like(coracc)
        acc1[...] = jnp.zeros_like(acc1)
        acc2[...] = jnp.zeros_like(acc2)
        acc3[...] = jnp.zeros_like(acc3)

    @pl.when(s == 0)
    def _stage0():
        onesr = jnp.ones((1, _BN), jnp.float32)
        lb = logits_ref[...]
        bp = jnp.clip(jax.nn.sigmoid(lb), _SC, 1.0 - _SC)
        onem = 1.0 - bp
        l1m = jnp.log(onem)
        fneg = -l1m * bp * bp
        negacc[...] += lax.dot_general(
            onesr, fneg, _DN_STD, preferred_element_type=jnp.float32)
        iib = iib_ref[...]
        iibf = iib.astype(jnp.float32)
        mask = iib > 0
        kk = lax.dot_general(iibf, jnp.ones((_G, 1), jnp.float32), _DN_STD,
                             preferred_element_type=jnp.float32)
        b = -fneg[:, 0:1]
        logp0 = jnp.clip(lb[:, 0:1] + l1m[:, 0:1],
                         jnp.log(_SC), jnp.log(1.0 - _SC))
        a = logp0 * onem[:, 0:1] * onem[:, 0:1]
        selb = jnp.where(kk > 0.0, b, 0.0)
        coracc[...] += lax.dot_general(
            onesr, selb, _DN_STD, preferred_element_type=jnp.float32)
        ab2 = jnp.concatenate([a, b], axis=1)
        ab[pl.ds(i * _BN, _BN), :] = ab2
        acc3[...] += lax.dot_general(
            ab2, iibf, _DN_TT, preferred_element_type=jnp.float32)
        align = lax.dot_general(scores_ref[...], oh_ref[...], _DN_STD,
                                preferred_element_type=jnp.float32) * iibf
        conf[...] = jnp.maximum(conf[...],
                                jnp.max(align, axis=0, keepdims=True))
        la[pl.ds(i * _BN, _BN), :] = jnp.where(
            mask, jnp.log(jnp.maximum(align, 1e-38)), 1.0)

    @pl.when(s == 1)
    def _stage1():
        giou = lax.dot_general(iou_ref[...], oh_ref[...], _DN_STD,
                               preferred_element_type=jnp.float32)
        lab = la[pl.ds(i * _BN, _BN), :]
        maskb = lab < 0.5
        p = jnp.exp(conf[...] * lab)
        q = p * giou + _SC
        mq = jnp.where(maskb, q, 0.0)
        mq2 = mq * q
        macc[...] = jnp.maximum(macc[...],
                                jnp.max(mq, axis=0, keepdims=True))
        abb = ab[pl.ds(i * _BN, _BN), :]
        acc1[...] += lax.dot_general(
            abb, mq2, _DN_TT, preferred_element_type=jnp.float32)
        acc2[...] += lax.dot_general(
            abb, mq, _DN_TT, preferred_element_type=jnp.float32)

    @pl.when((s == 1) & (i == _NB - 1))
    def _final():
        rinv = 1.0 / jnp.maximum(macc[...], _SC)
        rinv2 = rinv * rinv
        posv = acc1[0:1, :] * rinv2
        bnegv = (acc3[1:2, :] - 2.0 * acc2[1:2, :] * rinv
                 + acc1[1:2, :] * rinv2)
        sneg = jnp.sum(negacc[...], axis=1, keepdims=True)
        spos = jnp.sum(posv, axis=1, keepdims=True)
        sbneg = jnp.sum(bnegv, axis=1, keepdims=True)
        total = ((1.0 - _ALPHA) * (sneg + coracc[...])
                 - _ALPHA * (spos + sbneg))
        out_ref[...] = jnp.broadcast_to(total, out_ref.shape)


def kernel(logits_pred, scores, iou_map, is_in_boxes, labels, num_pos_avg):
    oh = (labels[None, :] ==
          jax.lax.broadcasted_iota(jnp.int32, (_C, _G), 0)).astype(jnp.float32)
    out = pl.pallas_call(
        _mask_loss_kernel,
        grid=(2, _NB),
        in_specs=[
            pl.BlockSpec((_C, _G), lambda s, i: (0, 0)),
            pl.BlockSpec((_BN, _C), lambda s, i: (jnp.where(s == 0, i, 0), 0)),
            pl.BlockSpec((_BN, _C), lambda s, i: (jnp.where(s == 0, i, 0), 0)),
            pl.BlockSpec((_BN, _G), lambda s, i: (jnp.where(s == 0, i, 0), 0)),
            pl.BlockSpec((_BN, _C), lambda s, i: (jnp.where(s == 1, i, 0), 0)),
        ],
        out_specs=pl.BlockSpec((8, 128), lambda s, i: (0, 0)),
        out_shape=jax.ShapeDtypeStruct((8, 128), jnp.float32),
        scratch_shapes=[
            pltpu.VMEM((_N, _G), jnp.float32),
            pltpu.VMEM((_N, 2), jnp.float32),
            pltpu.VMEM((1, _G), jnp.float32),
            pltpu.VMEM((1, _G), jnp.float32),
            pltpu.VMEM((1, _C), jnp.float32),
            pltpu.VMEM((1, 1), jnp.float32),
            pltpu.VMEM((2, _G), jnp.float32),
            pltpu.VMEM((2, _G), jnp.float32),
            pltpu.VMEM((2, _G), jnp.float32),
        ],
        compiler_params=pltpu.CompilerParams(
            dimension_semantics=("arbitrary", "arbitrary")),
    )(oh, logits_pred, scores, is_in_boxes, iou_map)
    return out[0, 0] / num_pos_avg


# speedup vs baseline: 1.7553x; 1.2320x over previous
import jax
import jax.numpy as jnp
from jax import lax
from jax.experimental import pallas as pl
from jax.experimental.pallas import tpu as pltpu

_GAMMA = 2.0
_SC = 0.0001
_ALPHA = 0.25

_N = 20000
_C = 80
_G = 128
_BN = 2000
_NB = _N // _BN

_DN_STD = (((1,), (0,)), ((), ()))
_DN_TT = (((0,), (0,)), ((), ()))


def _mask_loss_kernel(oh_ref, logits_ref, scores_ref, iib_ref, iou_ref,
                      out_ref, la, ab, conf, macc, negacc, coracc,
                      acc1, acc2, acc3):
    s = pl.program_id(0)
    i = pl.program_id(1)

    @pl.when((s == 0) & (i == 0))
    def _init():
        conf[...] = jnp.zeros_like(conf)
        macc[...] = jnp.zeros_like(macc)
        negacc[...] = jnp.zeros_like(negacc)
        coracc[...] = jnp.zeros_like(coracc)
        acc1[...] = jnp.zeros_like(acc1)
        acc2[...] = jnp.zeros_like(acc2)
        acc3[...] = jnp.zeros_like(acc3)

    @pl.when(s == 0)
    def _stage0():
        onesr = jnp.ones((1, _BN), jnp.float32)
        lb = logits_ref[...]
        bp = jnp.clip(jax.nn.sigmoid(lb), _SC, 1.0 - _SC)
        onem = 1.0 - bp
        l1m = jnp.log(onem)
        fneg = -l1m * bp * bp
        negacc[...] += lax.dot_general(
            onesr, fneg, _DN_STD, preferred_element_type=jnp.float32)
        iib = iib_ref[...]
        iibf = iib.astype(jnp.float32)
        mask = iib > 0
        kk = lax.dot_general(iibf, jnp.ones((_G, 1), jnp.float32), _DN_STD,
                             preferred_element_type=jnp.float32)
        b = -fneg[:, 0:1]
        logp0 = jnp.clip(lb[:, 0:1] + l1m[:, 0:1],
                         jnp.log(_SC), jnp.log(1.0 - _SC))
        a = logp0 * onem[:, 0:1] * onem[:, 0:1]
        selb = jnp.where(kk > 0.0, b, 0.0)
        coracc[...] += lax.dot_general(
            onesr, selb, _DN_STD, preferred_element_type=jnp.float32)
        ab2 = jnp.concatenate([a, b], axis=1)
        ab[pl.ds(i * _BN, _BN), :] = ab2
        acc3[...] += lax.dot_general(
            ab2, iibf, _DN_TT, preferred_element_type=jnp.float32)
        align = lax.dot_general(scores_ref[...], oh_ref[...], _DN_STD,
                                preferred_element_type=jnp.float32) * iibf
        conf[...] = jnp.maximum(conf[...],
                                jnp.max(align, axis=0, keepdims=True))
        la[pl.ds(i * _BN, _BN), :] = jnp.where(
            mask, jnp.log(jnp.maximum(align, 1e-38)), 1.0)

    @pl.when(s == 1)
    def _stage1():
        giou = lax.dot_general(iou_ref[...], oh_ref[...], _DN_STD,
                               preferred_element_type=jnp.float32)
        lab = la[pl.ds(i * _BN, _BN), :]
        maskb = lab < 0.5
        p = jnp.exp(conf[...] * lab)
        q = p * giou + _SC
        mq = jnp.where(maskb, q, 0.0)
        mq2 = mq * q
        macc[...] = jnp.maximum(macc[...],
                                jnp.max(mq, axis=0, keepdims=True))
        abb = ab[pl.ds(i * _BN, _BN), :]
        acc1[...] += lax.dot_general(
            abb, mq2, _DN_TT, preferred_element_type=jnp.float32)
        acc2[...] += lax.dot_general(
            abb, mq, _DN_TT, preferred_element_type=jnp.float32)

    @pl.when((s == 1) & (i == _NB - 1))
    def _final():
        rinv = 1.0 / jnp.maximum(macc[...], _SC)
        rinv2 = rinv * rinv
        posv = acc1[0:1, :] * rinv2
        bnegv = (acc3[1:2, :] - 2.0 * acc2[1:2, :] * rinv
                 + acc1[1:2, :] * rinv2)
        sneg = jnp.sum(negacc[...], axis=1, keepdims=True)
        spos = jnp.sum(posv, axis=1, keepdims=True)
        sbneg = jnp.sum(bnegv, axis=1, keepdims=True)
        total = ((1.0 - _ALPHA) * (sneg + coracc[...])
                 - _ALPHA * (spos + sbneg))
        out_ref[...] = jnp.broadcast_to(total, out_ref.shape)


def kernel(logits_pred, scores, iou_map, is_in_boxes, labels, num_pos_avg):
    oh = (labels[None, :] ==
          jax.lax.broadcasted_iota(jnp.int32, (_C, _G), 0)).astype(jnp.float32)
    out = pl.pallas_call(
        _mask_loss_kernel,
        grid=(2, _NB),
        in_specs=[
            pl.BlockSpec((_C, _G), lambda s, i: (0, 0)),
            pl.BlockSpec((_BN, _C), lambda s, i: (jnp.where(s == 0, i, 0), 0)),
            pl.BlockSpec((_BN, _C), lambda s, i: (jnp.where(s == 0, i, 0), 0)),
            pl.BlockSpec((_BN, _G), lambda s, i: (jnp.where(s == 0, i, 0), 0)),
            pl.BlockSpec((_BN, _C), lambda s, i: (jnp.where(s == 1, i, 0), 0)),
        ],
        out_specs=pl.BlockSpec((8, 128), lambda s, i: (0, 0)),
        out_shape=jax.ShapeDtypeStruct((8, 128), jnp.float32),
        scratch_shapes=[
            pltpu.VMEM((_N, _G), jnp.float32),
            pltpu.VMEM((_N, 2), jnp.float32),
            pltpu.VMEM((1, _G), jnp.float32),
            pltpu.VMEM((1, _G), jnp.float32),
            pltpu.VMEM((1, _C), jnp.float32),
            pltpu.VMEM((1, 1), jnp.float32),
            pltpu.VMEM((2, _G), jnp.float32),
            pltpu.VMEM((2, _G), jnp.float32),
            pltpu.VMEM((2, _G), jnp.float32),
        ],
        compiler_params=pltpu.CompilerParams(
            dimension_semantics=("arbitrary", "arbitrary")),
    )(oh, logits_pred, scores, is_in_boxes, iou_map)
    return out[0, 0] / num_pos_avg


# speedup vs baseline: 1.8662x; 1.0632x over previous
import jax
import jax.numpy as jnp
from jax import lax
from jax.experimental import pallas as pl
from jax.experimental.pallas import tpu as pltpu

_GAMMA = 2.0
_SC = 0.0001
_ALPHA = 0.25

_N = 20000
_C = 80
_G = 128
_BN = 4000
_NB = _N // _BN

_DN_STD = (((1,), (0,)), ((), ()))
_DN_TT = (((0,), (0,)), ((), ()))


def _mask_loss_kernel(oh_ref, logits_ref, scores_ref, iib_ref, iou_ref,
                      out_ref, la, ab, conf, macc, negacc, coracc,
                      acc1, acc2, acc3):
    s = pl.program_id(0)
    i = pl.program_id(1)

    @pl.when((s == 0) & (i == 0))
    def _init():
        conf[...] = jnp.zeros_like(conf)
        macc[...] = jnp.zeros_like(macc)
        negacc[...] = jnp.zeros_like(negacc)
        coracc[...] = jnp.zeros_like(coracc)
        acc1[...] = jnp.zeros_like(acc1)
        acc2[...] = jnp.zeros_like(acc2)
        acc3[...] = jnp.zeros_like(acc3)

    @pl.when(s == 0)
    def _stage0():
        onesr = jnp.ones((1, _BN), jnp.float32)
        lb = logits_ref[...]
        bp = jnp.clip(jax.nn.sigmoid(lb), _SC, 1.0 - _SC)
        onem = 1.0 - bp
        l1m = jnp.log(onem)
        fneg = -l1m * bp * bp
        negacc[...] += lax.dot_general(
            onesr, fneg, _DN_STD, preferred_element_type=jnp.float32)
        iib = iib_ref[...]
        iibf = iib.astype(jnp.float32)
        mask = iib > 0
        kk = lax.dot_general(iibf, jnp.ones((_G, 1), jnp.float32), _DN_STD,
                             preferred_element_type=jnp.float32)
        b = -fneg[:, 0:1]
        logp0 = jnp.clip(lb[:, 0:1] + l1m[:, 0:1],
                         jnp.log(_SC), jnp.log(1.0 - _SC))
        a = logp0 * onem[:, 0:1] * onem[:, 0:1]
        selb = jnp.where(kk > 0.0, b, 0.0)
        coracc[...] += lax.dot_general(
            onesr, selb, _DN_STD, preferred_element_type=jnp.float32)
        ab2 = jnp.concatenate([a, b], axis=1)
        ab[pl.ds(i * _BN, _BN), :] = ab2
        acc3[...] += lax.dot_general(
            ab2, iibf, _DN_TT, preferred_element_type=jnp.float32)
        align = lax.dot_general(scores_ref[...], oh_ref[...], _DN_STD,
                                preferred_element_type=jnp.float32) * iibf
        conf[...] = jnp.maximum(conf[...],
                                jnp.max(align, axis=0, keepdims=True))
        la[pl.ds(i * _BN, _BN), :] = jnp.where(
            mask, jnp.log(jnp.maximum(align, 1e-38)), 1.0)

    @pl.when(s == 1)
    def _stage1():
        giou = lax.dot_general(iou_ref[...], oh_ref[...], _DN_STD,
                               preferred_element_type=jnp.float32)
        lab = la[pl.ds(i * _BN, _BN), :]
        maskb = lab < 0.5
        p = jnp.exp(conf[...] * lab)
        q = p * giou + _SC
        mq = jnp.where(maskb, q, 0.0)
        mq2 = mq * q
        macc[...] = jnp.maximum(macc[...],
                                jnp.max(mq, axis=0, keepdims=True))
        abb = ab[pl.ds(i * _BN, _BN), :]
        acc1[...] += lax.dot_general(
            abb, mq2, _DN_TT, preferred_element_type=jnp.float32)
        acc2[...] += lax.dot_general(
            abb, mq, _DN_TT, preferred_element_type=jnp.float32)

    @pl.when((s == 1) & (i == _NB - 1))
    def _final():
        rinv = 1.0 / jnp.maximum(macc[...], _SC)
        rinv2 = rinv * rinv
        posv = acc1[0:1, :] * rinv2
        bnegv = (acc3[1:2, :] - 2.0 * acc2[1:2, :] * rinv
                 + acc1[1:2, :] * rinv2)
        sneg = jnp.sum(negacc[...], axis=1, keepdims=True)
        spos = jnp.sum(posv, axis=1, keepdims=True)
        sbneg = jnp.sum(bnegv, axis=1, keepdims=True)
        total = ((1.0 - _ALPHA) * (sneg + coracc[...])
                 - _ALPHA * (spos + sbneg))
        out_ref[...] = jnp.broadcast_to(total, out_ref.shape)


def kernel(logits_pred, scores, iou_map, is_in_boxes, labels, num_pos_avg):
    oh = (labels[None, :] ==
          jax.lax.broadcasted_iota(jnp.int32, (_C, _G), 0)).astype(jnp.float32)
    out = pl.pallas_call(
        _mask_loss_kernel,
        grid=(2, _NB),
        in_specs=[
            pl.BlockSpec((_C, _G), lambda s, i: (0, 0)),
            pl.BlockSpec((_BN, _C), lambda s, i: (jnp.where(s == 0, i, 0), 0)),
            pl.BlockSpec((_BN, _C), lambda s, i: (jnp.where(s == 0, i, 0), 0)),
            pl.BlockSpec((_BN, _G), lambda s, i: (jnp.where(s == 0, i, 0), 0)),
            pl.BlockSpec((_BN, _C), lambda s, i: (jnp.where(s == 1, i, 0), 0)),
        ],
        out_specs=pl.BlockSpec((8, 128), lambda s, i: (0, 0)),
        out_shape=jax.ShapeDtypeStruct((8, 128), jnp.float32),
        scratch_shapes=[
            pltpu.VMEM((_N, _G), jnp.float32),
            pltpu.VMEM((_N, 2), jnp.float32),
            pltpu.VMEM((1, _G), jnp.float32),
            pltpu.VMEM((1, _G), jnp.float32),
            pltpu.VMEM((1, _C), jnp.float32),
            pltpu.VMEM((1, 1), jnp.float32),
            pltpu.VMEM((2, _G), jnp.float32),
            pltpu.VMEM((2, _G), jnp.float32),
            pltpu.VMEM((2, _G), jnp.float32),
        ],
        compiler_params=pltpu.CompilerParams(
            dimension_semantics=("arbitrary", "arbitrary")),
    )(oh, logits_pred, scores, is_in_boxes, iou_map)
    return out[0, 0] / num_pos_avg
